# baseline (speedup 1.0000x reference)
import functools

import jax
import jax.numpy as jnp
from jax import lax
from jax.experimental import pallas as pl
from jax.experimental.pallas import tpu as pltpu
from jax.experimental.pallas import tpu_sc as plsc

B = 4
N = 8192
EMB = 768
ROWS = B * N
NC = 2
NS = 16
NW = NC * NS
RPW = ROWS // NW
K = 16
NCHUNK = RPW // K
NBUF = 4
LANES = 16
CPV = EMB // LANES

_mesh = plsc.VectorSubcoreMesh(core_axis_name="c", subcore_axis_name="s")


@functools.partial(
    pl.kernel,
    mesh=_mesh,
    out_type=jax.ShapeDtypeStruct((ROWS, EMB), jnp.float32),
    scratch_types=(
        [pltpu.VMEM((RPW,), jnp.int32)]
        + [pltpu.VMEM((K, EMB), jnp.float32)] * (2 * NBUF)
        + [pltpu.SemaphoreType.DMA] * (3 * NBUF)
    ),
)
def _pos_emb_sc(x_hbm, idx_hbm, tab_hbm, out_hbm, idx_v, *bufs_and_sems):
    gbufs = list(bufs_and_sems[0:NBUF])
    xbufs = list(bufs_and_sems[NBUF:2 * NBUF])
    rest = bufs_and_sems[2 * NBUF:]
    gsems = list(rest[0:NBUF])
    xsems = list(rest[NBUF:2 * NBUF])
    osems = list(rest[2 * NBUF:3 * NBUF])

    wid = lax.axis_index("s") * NC + lax.axis_index("c")
    base = wid * RPW
    pltpu.sync_copy(idx_hbm.at[pl.ds(base, RPW)], idx_v)

    def start_gather(g, b):
        pltpu.async_copy(tab_hbm.at[idx_v.at[pl.ds(g * K, K)]], gbufs[b],
                         gsems[b])

    def start_x(g, b):
        pltpu.async_copy(x_hbm.at[pl.ds(base + g * K, K)], xbufs[b],
                         xsems[b])

    def wait_loads(b):
        pltpu.make_async_copy(tab_hbm.at[idx_v.at[pl.ds(0, K)]], gbufs[b],
                              gsems[b]).wait()
        pltpu.make_async_copy(x_hbm.at[pl.ds(base, K)], xbufs[b],
                              xsems[b]).wait()

    def wait_out(b):
        pltpu.make_async_copy(xbufs[b], out_hbm.at[pl.ds(base, K)],
                              osems[b]).wait()

    def compute(b):
        gb, xb = gbufs[b], xbufs[b]

        def row_body(r, carry):
            for c in range(CPV):
                sl = pl.ds(c * LANES, LANES)
                plsc.addupdate(xb.at[r, sl], gb[r, sl])
            return carry
        lax.fori_loop(0, K, row_body, 0, unroll=2)

    start_gather(0, 0)
    start_x(0, 0)

    def group_body(i, carry):
        for p in range(NBUF):
            c = NBUF * i + p
            nb = (p + 1) % NBUF

            @pl.when(c + 1 < NCHUNK)
            def _():
                start_gather(c + 1, nb)

            @pl.when(jnp.logical_and(c >= NBUF - 1, c + 1 < NCHUNK))
            def _():
                wait_out(nb)

            @pl.when(c + 1 < NCHUNK)
            def _():
                start_x(c + 1, nb)
            wait_loads(p)
            compute(p)
            pltpu.async_copy(xbufs[p], out_hbm.at[pl.ds(base + c * K, K)],
                             osems[p])
        return carry

    lax.fori_loop(0, NCHUNK // NBUF, group_body, 0)
    for b in range(NBUF):
        wait_out(b)


def kernel(x, apply_indices, pos_embedding):
    xf = x.reshape(ROWS, EMB)
    idx = apply_indices.reshape(ROWS).astype(jnp.int32)
    out = _pos_emb_sc(xf, idx, pos_embedding)
    return out.reshape(x.shape)
